# Initial kernel scaffold; baseline (speedup 1.0000x reference)
import jax
import jax.numpy as jnp
from jax.experimental import pallas as pl


def kernel(x, Wd, bd, Wv, bv, Wu, bu, Wa, ba, Wc, bc, Ws, bs):
    raise NotImplementedError("write your pallas kernel here")


# speedup vs baseline: 3.2589x; 3.2589x over previous
import functools

import jax
import jax.numpy as jnp
from jax.experimental import pallas as pl
from jax.experimental.pallas import tpu as pltpu

_BLK = 1024


def _fused_kernel(n_blocks, gate_cols,
                  x_ref, wd_ref, bd_ref, wvu_ref, bvu_ref, wat_ref, ba_ref,
                  wc0_ref, wc1_ref, bcr_ref, ws0_ref, ws1_ref, bs_ref, s_ref,
                  aout_ref, outs_ref, bags_ref,
                  m_ref, l_ref, acc_ref):
    nb = pl.program_id(0)

    @pl.when(nb == 0)
    def _init():
        m_ref[...] = jnp.full_like(m_ref, -1e30)
        l_ref[...] = jnp.zeros_like(l_ref)
        acc_ref[...] = jnp.zeros_like(acc_ref)

    xb = x_ref[...]
    h = jnp.maximum(
        jnp.dot(xb, wd_ref[...], preferred_element_type=jnp.float32)
        + bd_ref[...], 0.0)
    g = jnp.dot(h, wvu_ref[...], preferred_element_type=jnp.float32) \
        + bvu_ref[...]
    gate = jnp.tanh(g[:, :gate_cols]) * jax.nn.sigmoid(g[:, gate_cols:])
    a_t = jax.lax.dot_general(
        wat_ref[...], gate, (((1,), (1,)), ((), ())),
        preferred_element_type=jnp.float32) + ba_ref[...]
    aout_ref[...] = a_t

    bm = jnp.max(a_t, axis=1, keepdims=True)
    m_old = m_ref[...]
    m_new = jnp.maximum(m_old, bm)
    corr = jnp.exp(m_old - m_new)
    p = jnp.exp(a_t - m_new)
    l_ref[...] = l_ref[...] * corr + jnp.sum(p, axis=1, keepdims=True)
    acc_ref[...] = acc_ref[...] * corr + jnp.dot(
        p, h, preferred_element_type=jnp.float32)
    m_ref[...] = m_new

    @pl.when(nb == n_blocks - 1)
    def _finish():
        afeat = acc_ref[...] / l_ref[...]
        o0 = jnp.sum(afeat * wc0_ref[...], axis=1, keepdims=True)
        o1 = jnp.sum(afeat * wc1_ref[...], axis=1, keepdims=True)
        outs_ref[...] = jnp.concatenate([o0, o1], axis=1) + bcr_ref[...]
        bag = jnp.dot(s_ref[...], afeat,
                      preferred_element_type=jnp.float32)
        b0 = jnp.sum(bag * ws0_ref[...], axis=1, keepdims=True)
        b1 = jnp.sum(bag * ws1_ref[...], axis=1, keepdims=True)
        bags_ref[...] = jnp.concatenate([b0, b1], axis=1) + bs_ref[...]


@jax.jit
def kernel(x, Wd, bd, Wv, bv, Wu, bu, Wa, ba, Wc, bc, Ws, bs):
    n = x.shape[1]
    d_feat = x.shape[2]
    d_inner = Wd.shape[1]
    n_task, _, d_att = Wv.shape
    n_token = Wa.shape[2]
    n_class = Wc.shape[3]
    r = n_task * n_token
    gate_cols = n_task * d_att
    n_blocks = n // _BLK

    x2 = x[0]
    Wvu = jnp.concatenate(
        [Wv.transpose(1, 0, 2).reshape(d_inner, gate_cols),
         Wu.transpose(1, 0, 2).reshape(d_inner, gate_cols)], axis=1)
    bvu = jnp.concatenate([bv.reshape(-1), bu.reshape(-1)])[None, :]
    eye_t = jnp.eye(n_task, dtype=Wa.dtype)
    WaT = jnp.einsum('ikj,im->ijmk', Wa, eye_t).reshape(r, gate_cols)
    ba_col = ba.reshape(r, 1)
    Wc_r = Wc.reshape(r, d_inner, n_class)
    bc_r = bc.reshape(r, n_class)
    S = jnp.repeat(jnp.eye(n_task, dtype=x.dtype), n_token, axis=1) / n_token

    body = functools.partial(_fused_kernel, n_blocks, gate_cols)
    aout, outs, bags = pl.pallas_call(
        body,
        grid=(n_blocks,),
        in_specs=[
            pl.BlockSpec((_BLK, d_feat), lambda nb: (nb, 0)),
            pl.BlockSpec((d_feat, d_inner), lambda nb: (0, 0)),
            pl.BlockSpec((1, d_inner), lambda nb: (0, 0)),
            pl.BlockSpec((d_inner, 2 * gate_cols), lambda nb: (0, 0)),
            pl.BlockSpec((1, 2 * gate_cols), lambda nb: (0, 0)),
            pl.BlockSpec((r, gate_cols), lambda nb: (0, 0)),
            pl.BlockSpec((r, 1), lambda nb: (0, 0)),
            pl.BlockSpec((r, d_inner), lambda nb: (0, 0)),
            pl.BlockSpec((r, d_inner), lambda nb: (0, 0)),
            pl.BlockSpec((r, n_class), lambda nb: (0, 0)),
            pl.BlockSpec((n_task, d_inner), lambda nb: (0, 0)),
            pl.BlockSpec((n_task, d_inner), lambda nb: (0, 0)),
            pl.BlockSpec((n_task, n_class), lambda nb: (0, 0)),
            pl.BlockSpec((n_task, r), lambda nb: (0, 0)),
        ],
        out_specs=[
            pl.BlockSpec((r, _BLK), lambda nb: (0, nb)),
            pl.BlockSpec((r, n_class), lambda nb: (0, 0)),
            pl.BlockSpec((n_task, n_class), lambda nb: (0, 0)),
        ],
        out_shape=[
            jax.ShapeDtypeStruct((r, n), jnp.float32),
            jax.ShapeDtypeStruct((r, n_class), jnp.float32),
            jax.ShapeDtypeStruct((n_task, n_class), jnp.float32),
        ],
        scratch_shapes=[
            pltpu.VMEM((r, 1), jnp.float32),
            pltpu.VMEM((r, 1), jnp.float32),
            pltpu.VMEM((r, d_inner), jnp.float32),
        ],
    )(x2, Wd, bd[None, :], Wvu, bvu, WaT, ba_col,
      Wc_r[:, :, 0], Wc_r[:, :, 1], bc_r, Ws[:, :, 0], Ws[:, :, 1], bs, S)

    outs_full = outs.reshape(n_task, n_token, n_class)
    bags_full = bags.reshape(n_task, 1, n_class)
    aouts_full = aout.reshape(n_task, n_token, n)[:, None, :, :]
    return outs_full, bags_full, aouts_full
